# baseline (speedup 1.0000x reference)
import functools

import jax
import jax.numpy as jnp
from jax import lax
from jax.experimental import pallas as pl
from jax.experimental.pallas import tpu as pltpu
from jax.experimental.pallas import tpu_sc as plsc

NNODES = 10000
DIM = 128
NEDGES = 320000

NCORES = 2
NSUB = 16
NTILES = NCORES * NSUB
BATCH = 128
NBATCH = 80
NHALF = NBATCH // 2
EPAD = NTILES * NBATCH * BATCH
NPAD = 10240
ROWS_PER_SUB = NPAD // NSUB
ZROWS = 32

ROW_BLK = 1000


def _sc_scatter_body(m_hbm, edges_hbm, out_hbm,
                     src_v, dst_v, r0, r1, zbuf, agg_sh, gsem, ssem):
    rows = (r0, r1)
    c = lax.axis_index("c")
    s = lax.axis_index("s")
    w = c * NSUB + s

    zero16 = jnp.zeros((16,), jnp.float32)

    def zrow(i, _):
        def zcol(j, _):
            zbuf[i, pl.ds(j * 16, 16)] = zero16
            return 0
        return lax.fori_loop(0, DIM // 16, zcol, 0)

    lax.fori_loop(0, ZROWS, zrow, 0)
    for k in range(ROWS_PER_SUB // ZROWS):
        pltpu.sync_copy(zbuf, agg_sh.at[pl.ds(s * ROWS_PER_SUB + k * ZROWS, ZROWS)])

    plsc.subcore_barrier()

    def fire_g(t, b):
        pltpu.async_copy(m_hbm.at[src_v.at[t]], rows[b], gsem)

    def wait_g(t, b):
        pltpu.make_async_copy(m_hbm.at[src_v.at[t]], rows[b], gsem).wait()

    def fire_s(t, b):
        pltpu.async_copy(rows[b], agg_sh.at[dst_v.at[t]], ssem, add=True)

    def wait_s(t, b):
        pltpu.make_async_copy(rows[b], agg_sh.at[dst_v.at[t]], ssem).wait()

    for half in range(2):
        pltpu.sync_copy(edges_hbm.at[0, w, pl.ds(half * NHALF, NHALF)], src_v)
        pltpu.sync_copy(edges_hbm.at[1, w, pl.ds(half * NHALF, NHALF)], dst_v)

        fire_g(0, 0)
        fire_g(1, 1)

        def slot(t, b, refill):
            wait_g(t, b)
            fire_s(t, b)
            wait_s(t, b)
            if refill:
                fire_g(t + 2, b)

        def body(i, _):
            slot(2 * i, 0, True)
            slot(2 * i + 1, 1, True)
            return 0

        lax.fori_loop(0, NHALF // 2 - 1, body, 0)
        slot(NHALF - 2, 0, False)
        slot(NHALF - 1, 1, False)

    plsc.subcore_barrier()

    pltpu.sync_copy(agg_sh.at[pl.ds(s * ROWS_PER_SUB, ROWS_PER_SUB)],
                    out_hbm.at[c, pl.ds(s * ROWS_PER_SUB, ROWS_PER_SUB)])


@functools.cache
def _make_sc_scatter():
    return pl.kernel(
        _sc_scatter_body,
        mesh=plsc.VectorSubcoreMesh(core_axis_name="c", subcore_axis_name="s",
                                    num_cores=NCORES, num_subcores=NSUB),
        out_type=jax.ShapeDtypeStruct((NCORES, NPAD, DIM), jnp.float32),
        scratch_types=[
            pltpu.VMEM((NHALF, BATCH), jnp.int32),
            pltpu.VMEM((NHALF, BATCH), jnp.int32),
            pltpu.VMEM((BATCH, DIM), jnp.float32),
            pltpu.VMEM((BATCH, DIM), jnp.float32),
            pltpu.VMEM((ZROWS, DIM), jnp.float32),
            pltpu.VMEM_SHARED((NPAD, DIM), jnp.float32),
            pltpu.SemaphoreType.DMA,
            pltpu.SemaphoreType.DMA,
        ],
        compiler_params=pltpu.CompilerParams(use_tc_tiling_on_sc=False),
    )


def _tc_m_body(h_ref, wi_ref, m_ref):
    m_ref[...] = jnp.dot(h_ref[...], wi_ref[...],
                         preferred_element_type=jnp.float32)


def _tc_m(h, wi):
    nblk = NNODES // ROW_BLK
    return pl.pallas_call(
        _tc_m_body,
        grid=(nblk,),
        in_specs=[
            pl.BlockSpec((ROW_BLK, DIM), lambda i: (i, 0)),
            pl.BlockSpec((DIM, DIM), lambda i: (0, 0)),
        ],
        out_specs=pl.BlockSpec((ROW_BLK, DIM), lambda i: (i, 0)),
        out_shape=jax.ShapeDtypeStruct((NNODES, DIM), jnp.float32),
    )(h, wi)


def _tc_gh_body(h_ref, whh_t_ref, bhh_ref, gh_ref):
    gh_ref[...] = (jnp.dot(h_ref[...], whh_t_ref[...],
                           preferred_element_type=jnp.float32)
                   + bhh_ref[...])


def _tc_gh(h, whh_t, bhh):
    nblk = NNODES // ROW_BLK
    return pl.pallas_call(
        _tc_gh_body,
        grid=(nblk,),
        in_specs=[
            pl.BlockSpec((ROW_BLK, DIM), lambda i: (i, 0)),
            pl.BlockSpec((DIM, 3 * DIM), lambda i: (0, 0)),
            pl.BlockSpec((1, 3 * DIM), lambda i: (0, 0)),
        ],
        out_specs=pl.BlockSpec((ROW_BLK, 3 * DIM), lambda i: (i, 0)),
        out_shape=jax.ShapeDtypeStruct((NNODES, 3 * DIM), jnp.float32),
    )(h, whh_t, bhh)


def _tc_gru_body(agg2_ref, gh_ref, h_ref, wih_t_ref, bih_ref, hout_ref):
    agg = agg2_ref[0] + agg2_ref[1]
    gi = (jnp.dot(agg, wih_t_ref[...], preferred_element_type=jnp.float32)
          + bih_ref[...])
    gh = gh_ref[...]
    h = h_ref[...]
    r = jax.nn.sigmoid(gi[:, :DIM] + gh[:, :DIM])
    z = jax.nn.sigmoid(gi[:, DIM:2 * DIM] + gh[:, DIM:2 * DIM])
    n = jnp.tanh(gi[:, 2 * DIM:] + r * gh[:, 2 * DIM:])
    hout_ref[...] = (1.0 - z) * n + z * h


def _tc_gru(agg2, gh, h, wih_t, bih):
    nblk = NNODES // ROW_BLK
    return pl.pallas_call(
        _tc_gru_body,
        grid=(nblk,),
        in_specs=[
            pl.BlockSpec((NCORES, ROW_BLK, DIM), lambda i: (0, i, 0)),
            pl.BlockSpec((ROW_BLK, 3 * DIM), lambda i: (i, 0)),
            pl.BlockSpec((ROW_BLK, DIM), lambda i: (i, 0)),
            pl.BlockSpec((DIM, 3 * DIM), lambda i: (0, 0)),
            pl.BlockSpec((1, 3 * DIM), lambda i: (0, 0)),
        ],
        out_specs=pl.BlockSpec((ROW_BLK, DIM), lambda i: (i, 0)),
        out_shape=jax.ShapeDtypeStruct((NNODES, DIM), jnp.float32),
    )(agg2, gh, h, wih_t, bih)


def _tc_gru_fin_body(agg2_ref, gh_ref, h_ref, wih_t_ref, bih_ref,
                     fcw_t_ref, fcb_ref, o_ref):
    agg = agg2_ref[0] + agg2_ref[1]
    gi = (jnp.dot(agg, wih_t_ref[...], preferred_element_type=jnp.float32)
          + bih_ref[...])
    gh = gh_ref[...]
    h = h_ref[...]
    r = jax.nn.sigmoid(gi[:, :DIM] + gh[:, :DIM])
    z = jax.nn.sigmoid(gi[:, DIM:2 * DIM] + gh[:, DIM:2 * DIM])
    n = jnp.tanh(gi[:, 2 * DIM:] + r * gh[:, 2 * DIM:])
    hnew = (1.0 - z) * n + z * h
    o_ref[...] = (jnp.dot(jnp.maximum(hnew, 0.0), fcw_t_ref[...],
                          preferred_element_type=jnp.float32)
                  + fcb_ref[...])


def _tc_gru_fin(agg2, gh, h, wih_t, bih, fcw_t, fcb):
    nblk = NNODES // ROW_BLK
    return pl.pallas_call(
        _tc_gru_fin_body,
        grid=(nblk,),
        in_specs=[
            pl.BlockSpec((NCORES, ROW_BLK, DIM), lambda i: (0, i, 0)),
            pl.BlockSpec((ROW_BLK, 3 * DIM), lambda i: (i, 0)),
            pl.BlockSpec((ROW_BLK, DIM), lambda i: (i, 0)),
            pl.BlockSpec((DIM, 3 * DIM), lambda i: (0, 0)),
            pl.BlockSpec((1, 3 * DIM), lambda i: (0, 0)),
            pl.BlockSpec((DIM, DIM), lambda i: (0, 0)),
            pl.BlockSpec((1, DIM), lambda i: (0, 0)),
        ],
        out_specs=pl.BlockSpec((ROW_BLK, DIM), lambda i: (i, 0)),
        out_shape=jax.ShapeDtypeStruct((NNODES, DIM), jnp.float32),
    )(agg2, gh, h, wih_t, bih, fcw_t, fcb)


def kernel(x, edge_index, W, w_ih, w_hh, b_ih, b_hh, fc_w, fc_b):
    src = edge_index[0]
    dst = edge_index[1]
    pad = EPAD - NEDGES
    src_p = jnp.concatenate([src, jnp.zeros((pad,), jnp.int32)])
    dst_p = jnp.concatenate([dst, jnp.full((pad,), NNODES, jnp.int32)])
    edges_p = jnp.stack([src_p, dst_p]).reshape(2, NTILES, NBATCH, BATCH)

    whh_t = w_hh.T
    wih_t = w_ih.T
    fcw_t = fc_w.T
    bhh = b_hh.reshape(1, -1)
    bih = b_ih.reshape(1, -1)
    fcb = fc_b.reshape(1, -1)

    nl = W.shape[0]
    h = x
    for i in range(nl):
        m = _tc_m(h, W[i])
        agg2 = _make_sc_scatter()(m, edges_p)
        gh = _tc_gh(h, whh_t, bhh)
        if i < nl - 1:
            h = _tc_gru(agg2, gh, h, wih_t, bih)
        else:
            h = _tc_gru_fin(agg2, gh, h, wih_t, bih, fcw_t, fcb)
    return h


# speedup vs baseline: 2.8842x; 2.8842x over previous
import functools

import jax
import jax.numpy as jnp
from jax import lax
from jax.experimental import pallas as pl
from jax.experimental.pallas import tpu as pltpu
from jax.experimental.pallas import tpu_sc as plsc

NNODES = 10000
DIM = 128
NEDGES = 320000

NCORES = 2
NSUB = 16
HDIM = DIM // NCORES
BATCH = 128
NBATCH = 160
NHALF = NBATCH // 2
EPAD = NSUB * NBATCH * BATCH
NPAD = 10240
ROWS_PER_SUB = NPAD // NSUB
MROWS_PER_SUB = NNODES // NSUB
ZROWS = 16

ROW_BLK = 1000


def _sc_scatter_body(m_hbm, edges_hbm, out_hbm,
                     src_v, dst_v, r0, r1, r2, zbuf, m_sp, agg_sh, gsem, ssem):
    rows = (r0, r1, r2)
    c = lax.axis_index("c")
    s = lax.axis_index("s")

    zero16 = jnp.zeros((16,), jnp.float32)

    def zrow(i, _):
        def zcol(j, _):
            zbuf[i, pl.ds(j * 16, 16)] = zero16
            return 0
        return lax.fori_loop(0, HDIM // 16, zcol, 0)

    lax.fori_loop(0, ZROWS, zrow, 0)
    for k in range(ROWS_PER_SUB // ZROWS):
        pltpu.sync_copy(zbuf, agg_sh.at[pl.ds(s * ROWS_PER_SUB + k * ZROWS, ZROWS)])
    pltpu.sync_copy(m_hbm.at[c, pl.ds(s * MROWS_PER_SUB, MROWS_PER_SUB)],
                    m_sp.at[pl.ds(s * MROWS_PER_SUB, MROWS_PER_SUB)])

    plsc.subcore_barrier()

    def fire_g(t, b):
        pltpu.async_copy(m_sp.at[src_v.at[t]], rows[b], gsem)

    def wait_g(t, b):
        pltpu.make_async_copy(m_sp.at[src_v.at[t]], rows[b], gsem).wait()

    def fire_s(t, b):
        pltpu.async_copy(rows[b], agg_sh.at[dst_v.at[t]], ssem, add=True)

    def wait_s(t, b):
        pltpu.make_async_copy(rows[b], agg_sh.at[dst_v.at[t]], ssem).wait()

    for half in range(2):
        pltpu.sync_copy(edges_hbm.at[0, s, pl.ds(half * NHALF, NHALF)], src_v)
        pltpu.sync_copy(edges_hbm.at[1, s, pl.ds(half * NHALF, NHALF)], dst_v)

        fire_g(0, 0)
        fire_g(1, 1)
        wait_g(0, 0)
        fire_s(0, 0)
        fire_g(2, 2)

        def body(i, _):
            for u in range(3):
                tu = 1 + 3 * i + u
                b = (1 + u) % 3
                wait_g(tu, b)
                fire_s(tu, b)
                wait_s(tu - 1, u % 3)
                fire_g(tu + 2, u % 3)
            return 0

        lax.fori_loop(0, 25, body, 0)
        for tu in range(76, NHALF):
            wait_g(tu, tu % 3)
            fire_s(tu, tu % 3)
            wait_s(tu - 1, (tu - 1) % 3)
            if tu + 2 < NHALF:
                fire_g(tu + 2, (tu + 2) % 3)
        wait_s(NHALF - 1, (NHALF - 1) % 3)

    plsc.subcore_barrier()

    pltpu.sync_copy(agg_sh.at[pl.ds(s * ROWS_PER_SUB, ROWS_PER_SUB)],
                    out_hbm.at[c, pl.ds(s * ROWS_PER_SUB, ROWS_PER_SUB)])


@functools.cache
def _make_sc_scatter():
    return pl.kernel(
        _sc_scatter_body,
        mesh=plsc.VectorSubcoreMesh(core_axis_name="c", subcore_axis_name="s",
                                    num_cores=NCORES, num_subcores=NSUB),
        out_type=jax.ShapeDtypeStruct((NCORES, NPAD, HDIM), jnp.float32),
        scratch_types=[
            pltpu.VMEM((NHALF, BATCH), jnp.int32),
            pltpu.VMEM((NHALF, BATCH), jnp.int32),
            pltpu.VMEM((BATCH, HDIM), jnp.float32),
            pltpu.VMEM((BATCH, HDIM), jnp.float32),
            pltpu.VMEM((BATCH, HDIM), jnp.float32),
            pltpu.VMEM((ZROWS, HDIM), jnp.float32),
            pltpu.VMEM_SHARED((NNODES, HDIM), jnp.float32),
            pltpu.VMEM_SHARED((NPAD, HDIM), jnp.float32),
            pltpu.SemaphoreType.DMA,
            pltpu.SemaphoreType.DMA,
        ],
        compiler_params=pltpu.CompilerParams(use_tc_tiling_on_sc=False),
    )


def _tc_m_body(h_ref, wi_ref, m_ref):
    h = h_ref[...]
    m = jnp.dot(h, wi_ref[...], preferred_element_type=jnp.float32)
    m_ref[0] = m[:, :HDIM]
    m_ref[1] = m[:, HDIM:]


def _tc_m(h, wi):
    nblk = NNODES // ROW_BLK
    return pl.pallas_call(
        _tc_m_body,
        grid=(nblk,),
        in_specs=[
            pl.BlockSpec((ROW_BLK, DIM), lambda i: (i, 0)),
            pl.BlockSpec((DIM, DIM), lambda i: (0, 0)),
        ],
        out_specs=pl.BlockSpec((NCORES, ROW_BLK, HDIM), lambda i: (0, i, 0)),
        out_shape=jax.ShapeDtypeStruct((NCORES, NNODES, HDIM), jnp.float32),
    )(h, wi)


def _tc_gh_body(h_ref, whh_t_ref, bhh_ref, gh_ref):
    gh_ref[...] = (jnp.dot(h_ref[...], whh_t_ref[...],
                           preferred_element_type=jnp.float32)
                   + bhh_ref[...])


def _tc_gh(h, whh_t, bhh):
    nblk = NNODES // ROW_BLK
    return pl.pallas_call(
        _tc_gh_body,
        grid=(nblk,),
        in_specs=[
            pl.BlockSpec((ROW_BLK, DIM), lambda i: (i, 0)),
            pl.BlockSpec((DIM, 3 * DIM), lambda i: (0, 0)),
            pl.BlockSpec((1, 3 * DIM), lambda i: (0, 0)),
        ],
        out_specs=pl.BlockSpec((ROW_BLK, 3 * DIM), lambda i: (i, 0)),
        out_shape=jax.ShapeDtypeStruct((NNODES, 3 * DIM), jnp.float32),
    )(h, whh_t, bhh)


def _tc_gru_body(agg2_ref, gh_ref, h_ref, wih_t_ref, bih_ref, hout_ref):
    agg = jnp.concatenate([agg2_ref[0], agg2_ref[1]], axis=1)
    gi = (jnp.dot(agg, wih_t_ref[...], preferred_element_type=jnp.float32)
          + bih_ref[...])
    gh = gh_ref[...]
    h = h_ref[...]
    r = jax.nn.sigmoid(gi[:, :DIM] + gh[:, :DIM])
    z = jax.nn.sigmoid(gi[:, DIM:2 * DIM] + gh[:, DIM:2 * DIM])
    n = jnp.tanh(gi[:, 2 * DIM:] + r * gh[:, 2 * DIM:])
    hout_ref[...] = (1.0 - z) * n + z * h


def _tc_gru(agg2, gh, h, wih_t, bih):
    nblk = NNODES // ROW_BLK
    return pl.pallas_call(
        _tc_gru_body,
        grid=(nblk,),
        in_specs=[
            pl.BlockSpec((NCORES, ROW_BLK, HDIM), lambda i: (0, i, 0)),
            pl.BlockSpec((ROW_BLK, 3 * DIM), lambda i: (i, 0)),
            pl.BlockSpec((ROW_BLK, DIM), lambda i: (i, 0)),
            pl.BlockSpec((DIM, 3 * DIM), lambda i: (0, 0)),
            pl.BlockSpec((1, 3 * DIM), lambda i: (0, 0)),
        ],
        out_specs=pl.BlockSpec((ROW_BLK, DIM), lambda i: (i, 0)),
        out_shape=jax.ShapeDtypeStruct((NNODES, DIM), jnp.float32),
    )(agg2, gh, h, wih_t, bih)


def _tc_gru_fin_body(agg2_ref, gh_ref, h_ref, wih_t_ref, bih_ref,
                     fcw_t_ref, fcb_ref, o_ref):
    agg = jnp.concatenate([agg2_ref[0], agg2_ref[1]], axis=1)
    gi = (jnp.dot(agg, wih_t_ref[...], preferred_element_type=jnp.float32)
          + bih_ref[...])
    gh = gh_ref[...]
    h = h_ref[...]
    r = jax.nn.sigmoid(gi[:, :DIM] + gh[:, :DIM])
    z = jax.nn.sigmoid(gi[:, DIM:2 * DIM] + gh[:, DIM:2 * DIM])
    n = jnp.tanh(gi[:, 2 * DIM:] + r * gh[:, 2 * DIM:])
    hnew = (1.0 - z) * n + z * h
    o_ref[...] = (jnp.dot(jnp.maximum(hnew, 0.0), fcw_t_ref[...],
                          preferred_element_type=jnp.float32)
                  + fcb_ref[...])


def _tc_gru_fin(agg2, gh, h, wih_t, bih, fcw_t, fcb):
    nblk = NNODES // ROW_BLK
    return pl.pallas_call(
        _tc_gru_fin_body,
        grid=(nblk,),
        in_specs=[
            pl.BlockSpec((NCORES, ROW_BLK, HDIM), lambda i: (0, i, 0)),
            pl.BlockSpec((ROW_BLK, 3 * DIM), lambda i: (i, 0)),
            pl.BlockSpec((ROW_BLK, DIM), lambda i: (i, 0)),
            pl.BlockSpec((DIM, 3 * DIM), lambda i: (0, 0)),
            pl.BlockSpec((1, 3 * DIM), lambda i: (0, 0)),
            pl.BlockSpec((DIM, DIM), lambda i: (0, 0)),
            pl.BlockSpec((1, DIM), lambda i: (0, 0)),
        ],
        out_specs=pl.BlockSpec((ROW_BLK, DIM), lambda i: (i, 0)),
        out_shape=jax.ShapeDtypeStruct((NNODES, DIM), jnp.float32),
    )(agg2, gh, h, wih_t, bih, fcw_t, fcb)


def kernel(x, edge_index, W, w_ih, w_hh, b_ih, b_hh, fc_w, fc_b):
    src = edge_index[0]
    dst = edge_index[1]
    pad = EPAD - NEDGES
    src_p = jnp.concatenate([src, jnp.zeros((pad,), jnp.int32)])
    dst_p = jnp.concatenate([dst, jnp.full((pad,), NNODES, jnp.int32)])
    edges_p = jnp.stack([src_p, dst_p]).reshape(2, NSUB, NBATCH, BATCH)

    whh_t = w_hh.T
    wih_t = w_ih.T
    fcw_t = fc_w.T
    bhh = b_hh.reshape(1, -1)
    bih = b_ih.reshape(1, -1)
    fcb = fc_b.reshape(1, -1)

    nl = W.shape[0]
    h = x
    for i in range(nl):
        m = _tc_m(h, W[i])
        agg2 = _make_sc_scatter()(m, edges_p)
        gh = _tc_gh(h, whh_t, bhh)
        if i < nl - 1:
            h = _tc_gru(agg2, gh, h, wih_t, bih)
        else:
            h = _tc_gru_fin(agg2, gh, h, wih_t, bih, fcw_t, fcb)
    return h
